# baseline (speedup 1.0000x reference)
import functools

import jax
import jax.numpy as jnp
from jax import lax
from jax.experimental import pallas as pl
from jax.experimental.pallas import tpu as pltpu
from jax.experimental.pallas import tpu_sc as plsc

NC = 2
NS = 16
NW = NC * NS
L = 16
IDX_CHUNK = 128


def _build(batch, dim, n_users, n_items):
    b_per_w = batch // NW
    n_chunks = b_per_w // IDX_CHUNK
    groups_per_chunk = IDX_CHUNK // L
    wdim = 2 * dim
    mesh = plsc.VectorSubcoreMesh(
        core_axis_name="c", subcore_axis_name="s", num_cores=NC, num_subcores=NS
    )

    @functools.partial(
        pl.kernel,
        mesh=mesh,
        out_type=jax.ShapeDtypeStruct((batch,), jnp.float32),
        compiler_params=pltpu.CompilerParams(
            needs_layout_passes=False, use_tc_tiling_on_sc=True),
        scratch_types=[
            pltpu.VMEM((b_per_w,), jnp.int32),
            pltpu.VMEM((b_per_w,), jnp.int32),
            pltpu.VMEM((n_chunks, IDX_CHUNK), jnp.int32),
            pltpu.VMEM((n_chunks, IDX_CHUNK), jnp.int32),
            pltpu.VMEM((n_chunks, IDX_CHUNK), jnp.int32),
            pltpu.VMEM((n_chunks, IDX_CHUNK), jnp.int32),
            pltpu.VMEM((b_per_w,), jnp.int32),
            pltpu.VMEM((b_per_w,), jnp.int32),
            pltpu.VMEM((2, IDX_CHUNK, wdim), jnp.float32),
            pltpu.VMEM((2, IDX_CHUNK, wdim), jnp.float32),
            pltpu.VMEM((b_per_w,), jnp.float32),
            pltpu.VMEM((b_per_w,), jnp.float32),
            pltpu.VMEM((b_per_w,), jnp.float32),
            pltpu.VMEM((b_per_w,), jnp.float32),
            pltpu.SemaphoreType.DMA,
            pltpu.SemaphoreType.DMA,
        ],
    )
    def mf(dense_hbm, uid_hbm, iid_hbm, pr_hbm, qr_hbm, ub_hbm, ib_hbm,
           out_hbm, uraw_v, iraw_v, upair_v, ipair_v, uw_v, iw_v, uoff_v,
           ioff_v, prows_v, qrows_v, dense_v, ub_v, ib_v, out_v, sem, bsem):
        wid = lax.axis_index("s") * NC + lax.axis_index("c")
        base = wid * b_per_w

        pltpu.sync_copy(uid_hbm.at[pl.ds(base, b_per_w)], uraw_v)
        pltpu.sync_copy(iid_hbm.at[pl.ds(base, b_per_w)], iraw_v)
        pltpu.sync_copy(dense_hbm.at[pl.ds(base, b_per_w)], dense_v)

        iota = lax.iota(jnp.int32, L)
        for j in range(b_per_w // L):
            sl = pl.ds(j * L, L)
            row, col = divmod(j * L, IDX_CHUNK)
            csl = pl.ds(col, L)
            u = uraw_v[sl]
            w = jnp.where(u == 0, n_users - 1, u - 1)
            j = w >> 11
            upair_v[row, csl] = ((j >> 1) << 11) + (w & 2047)
            uw_v[row, csl] = w
            uoff_v[sl] = (j & 1) << 6
            t = iraw_v[sl]
            w = jnp.where(t == 0, n_items - 1, t - 1)
            j = w >> 11
            ipair_v[row, csl] = ((j >> 1) << 11) + (w & 2047)
            iw_v[row, csl] = w
            ioff_v[sl] = (j & 1) << 6

        bias_copies = []
        for ck in range(n_chunks):
            sl = pl.ds(ck * IDX_CHUNK, IDX_CHUNK)
            bias_copies.append(
                pltpu.async_copy(ub_hbm.at[uw_v.at[ck]], ub_v.at[sl], bsem))
            bias_copies.append(
                pltpu.async_copy(ib_hbm.at[iw_v.at[ck]], ib_v.at[sl], bsem))

        def fire(ck, buf):
            cp = pltpu.async_copy(
                pr_hbm.at[upair_v.at[ck]], prows_v.at[buf], sem)
            cq = pltpu.async_copy(
                qr_hbm.at[ipair_v.at[ck]], qrows_v.at[buf], sem)
            return cp, cq

        inflight = fire(0, 0)
        for cp in bias_copies:
            cp.wait()

        for ck in range(n_chunks):
            buf = ck % 2
            cur = inflight
            if ck + 1 < n_chunks:
                inflight = fire(ck + 1, 1 - buf)
            cur[0].wait()
            cur[1].wait()

            def group(lg, carry):
                rid = iota + (ck * IDX_CHUNK + lg * L)
                lrid = iota + lg * L
                acc = (plsc.load_gather(dense_v, [rid])
                       + plsc.load_gather(ub_v, [rid])
                       + plsc.load_gather(ib_v, [rid]))
                uo = plsc.load_gather(uoff_v, [rid])
                io = plsc.load_gather(ioff_v, [rid])
                for d in range(dim):
                    pv = plsc.load_gather(prows_v, [jnp.full((L,), buf, jnp.int32),
                                                    lrid, uo + d])
                    qv = plsc.load_gather(qrows_v, [jnp.full((L,), buf, jnp.int32),
                                                    lrid, io + d])
                    acc = acc + pv * qv
                plsc.store_scatter(out_v, [rid], acc)
                return carry

            lax.fori_loop(0, groups_per_chunk, group, 0)

        pltpu.sync_copy(out_v, out_hbm.at[pl.ds(base, b_per_w)])

    return mf


def _build_pairify(n_rows, dim):
    blk = 2048
    grid = (n_rows + 2 * blk - 1) // (2 * blk)
    out_rows = grid * blk

    def body(xlo_ref, xhi_ref, o_ref):
        eye = jnp.eye(dim, dtype=jnp.float32)

        def t(x):
            return jax.lax.dot_general(
                x, eye, (((0,), (0,)), ((), ())),
                precision=jax.lax.Precision.HIGHEST)

        o_ref[:, 0:dim] = t(xlo_ref[...])
        o_ref[:, dim:2 * dim] = t(xhi_ref[...])

    return pl.pallas_call(
        body,
        grid=(grid,),
        in_specs=[pl.BlockSpec((dim, blk), lambda g: (0, 2 * g)),
                  pl.BlockSpec(
                      (dim, blk),
                      lambda g: (0, jnp.minimum(2 * g + 1, 2 * grid - 2)))],
        out_specs=pl.BlockSpec((blk, 2 * dim), lambda g: (g, 0)),
        out_shape=jax.ShapeDtypeStruct((out_rows, 2 * dim), jnp.float32),
    )


def kernel(dense_inputs, sparse_inputs, p, q, user_bias, item_bias):
    batch = sparse_inputs.shape[0]
    dim = p.shape[1]
    pairify = _build_pairify(p.shape[0], dim)
    pt = p.T
    qt = q.T
    pr = pairify(pt, pt)
    qr = pairify(qt, qt)
    mf = _build(batch, dim, p.shape[0], q.shape[0])
    out = mf(dense_inputs.reshape(-1),
             sparse_inputs[:, 0], sparse_inputs[:, 1],
             pr, qr,
             user_bias.reshape(-1), item_bias.reshape(-1))
    return out.reshape(batch, 1)


# speedup vs baseline: 1.5351x; 1.5351x over previous
import functools

import jax
import jax.numpy as jnp
from jax import lax
from jax.experimental import pallas as pl
from jax.experimental.pallas import tpu as pltpu
from jax.experimental.pallas import tpu_sc as plsc

NC = 2
NS = 16
NW = NC * NS
L = 16
IDX_CHUNK = 128


def _build(batch, dim, n_users, n_items):
    b_per_w = batch // NW
    n_chunks = b_per_w // IDX_CHUNK
    groups_per_chunk = IDX_CHUNK // L
    wdim = 2 * dim
    mesh = plsc.VectorSubcoreMesh(
        core_axis_name="c", subcore_axis_name="s", num_cores=NC, num_subcores=NS
    )

    @functools.partial(
        pl.kernel,
        mesh=mesh,
        out_type=jax.ShapeDtypeStruct((batch,), jnp.float32),
        compiler_params=pltpu.CompilerParams(
            needs_layout_passes=False, use_tc_tiling_on_sc=True),
        scratch_types=[
            pltpu.VMEM((b_per_w,), jnp.int32),
            pltpu.VMEM((b_per_w,), jnp.int32),
            pltpu.VMEM((n_chunks, IDX_CHUNK), jnp.int32),
            pltpu.VMEM((n_chunks, IDX_CHUNK), jnp.int32),
            pltpu.VMEM((n_chunks, IDX_CHUNK), jnp.int32),
            pltpu.VMEM((n_chunks, IDX_CHUNK), jnp.int32),
            pltpu.VMEM((b_per_w,), jnp.int32),
            pltpu.VMEM((b_per_w,), jnp.int32),
            pltpu.VMEM((2, IDX_CHUNK, wdim), jnp.float32),
            pltpu.VMEM((2, IDX_CHUNK, wdim), jnp.float32),
            pltpu.VMEM((b_per_w,), jnp.float32),
            pltpu.VMEM((b_per_w,), jnp.float32),
            pltpu.VMEM((b_per_w,), jnp.float32),
            pltpu.VMEM((b_per_w,), jnp.float32),
            pltpu.SemaphoreType.DMA,
            pltpu.SemaphoreType.DMA,
        ],
    )
    def mf(dense_hbm, uid_hbm, iid_hbm, pr_hbm, qr_hbm, ub_hbm, ib_hbm,
           out_hbm, uraw_v, iraw_v, upair_v, ipair_v, uw_v, iw_v, uoff_v,
           ioff_v, prows_v, qrows_v, dense_v, ub_v, ib_v, out_v, sem, bsem):
        wid = lax.axis_index("s") * NC + lax.axis_index("c")
        base = wid * b_per_w

        pltpu.sync_copy(uid_hbm.at[pl.ds(base, b_per_w)], uraw_v)
        pltpu.sync_copy(iid_hbm.at[pl.ds(base, b_per_w)], iraw_v)
        pltpu.sync_copy(dense_hbm.at[pl.ds(base, b_per_w)], dense_v)

        iota = lax.iota(jnp.int32, L)
        for j in range(b_per_w // L):
            sl = pl.ds(j * L, L)
            row, col = divmod(j * L, IDX_CHUNK)
            csl = pl.ds(col, L)
            u = uraw_v[sl]
            w = jnp.where(u == 0, n_users - 1, u - 1)
            j = w >> 11
            upair_v[row, csl] = ((j >> 1) << 11) + (w & 2047)
            uw_v[row, csl] = w
            uoff_v[sl] = (j & 1) << 6
            t = iraw_v[sl]
            w = jnp.where(t == 0, n_items - 1, t - 1)
            j = w >> 11
            ipair_v[row, csl] = ((j >> 1) << 11) + (w & 2047)
            iw_v[row, csl] = w
            ioff_v[sl] = (j & 1) << 6

        bias_copies = []
        for ck in range(n_chunks):
            sl = pl.ds(ck * IDX_CHUNK, IDX_CHUNK)
            bias_copies.append(
                pltpu.async_copy(ub_hbm.at[uw_v.at[ck]], ub_v.at[sl], bsem))
            bias_copies.append(
                pltpu.async_copy(ib_hbm.at[iw_v.at[ck]], ib_v.at[sl], bsem))

        def fire(ck, buf):
            cp = pltpu.async_copy(
                pr_hbm.at[upair_v.at[ck]], prows_v.at[buf], sem)
            cq = pltpu.async_copy(
                qr_hbm.at[ipair_v.at[ck]], qrows_v.at[buf], sem)
            return cp, cq

        inflight = fire(0, 0)
        for cp in bias_copies:
            cp.wait()

        for ck in range(n_chunks):
            buf = ck % 2
            cur = inflight
            if ck + 1 < n_chunks:
                inflight = fire(ck + 1, 1 - buf)
            cur[0].wait()
            cur[1].wait()

            def group(lg, carry):
                rid = iota + (ck * IDX_CHUNK + lg * L)
                lrid = iota + lg * L
                acc = (plsc.load_gather(dense_v, [rid])
                       + plsc.load_gather(ub_v, [rid])
                       + plsc.load_gather(ib_v, [rid]))
                uo = plsc.load_gather(uoff_v, [rid])
                io = plsc.load_gather(ioff_v, [rid])
                for d in range(dim):
                    pv = plsc.load_gather(prows_v, [jnp.full((L,), buf, jnp.int32),
                                                    lrid, uo + d])
                    qv = plsc.load_gather(qrows_v, [jnp.full((L,), buf, jnp.int32),
                                                    lrid, io + d])
                    acc = acc + pv * qv
                plsc.store_scatter(out_v, [rid], acc)
                return carry

            lax.fori_loop(0, groups_per_chunk, group, 0)

        pltpu.sync_copy(out_v, out_hbm.at[pl.ds(base, b_per_w)])

    return mf


def _build_pairify(n_rows, dim):
    blk = 2048
    grid = (n_rows + 2 * blk - 1) // (2 * blk)
    out_rows = grid * blk

    def body(xlo_ref, xhi_ref, o_ref):
        o_ref[:, 0:dim] = xlo_ref[...].T
        o_ref[:, dim:2 * dim] = xhi_ref[...].T

    return pl.pallas_call(
        body,
        grid=(grid,),
        in_specs=[pl.BlockSpec((dim, blk), lambda g: (0, 2 * g)),
                  pl.BlockSpec(
                      (dim, blk),
                      lambda g: (0, jnp.minimum(2 * g + 1, 2 * grid - 2)))],
        out_specs=pl.BlockSpec((blk, 2 * dim), lambda g: (g, 0)),
        out_shape=jax.ShapeDtypeStruct((out_rows, 2 * dim), jnp.float32),
    )


def kernel(dense_inputs, sparse_inputs, p, q, user_bias, item_bias):
    batch = sparse_inputs.shape[0]
    dim = p.shape[1]
    pairify = _build_pairify(p.shape[0], dim)
    pt = p.T
    qt = q.T
    pr = pairify(pt, pt)
    qr = pairify(qt, qt)
    mf = _build(batch, dim, p.shape[0], q.shape[0])
    out = mf(dense_inputs.reshape(-1),
             sparse_inputs[:, 0], sparse_inputs[:, 1],
             pr, qr,
             user_bias.reshape(-1), item_bias.reshape(-1))
    return out.reshape(batch, 1)


# speedup vs baseline: 1.8397x; 1.1985x over previous
import functools

import jax
import jax.numpy as jnp
from jax import lax
from jax.experimental import pallas as pl
from jax.experimental.pallas import tpu as pltpu
from jax.experimental.pallas import tpu_sc as plsc

NC = 2
NS = 16
NW = NC * NS
L = 16
IDX_CHUNK = 128


def _build(batch, dim, n_users, n_items):
    b_per_w = batch // NW
    n_chunks = b_per_w // IDX_CHUNK
    groups_per_chunk = IDX_CHUNK // L
    wdim = 2 * dim
    mesh = plsc.VectorSubcoreMesh(
        core_axis_name="c", subcore_axis_name="s", num_cores=NC, num_subcores=NS
    )

    @functools.partial(
        pl.kernel,
        mesh=mesh,
        out_type=jax.ShapeDtypeStruct((batch,), jnp.float32),
        compiler_params=pltpu.CompilerParams(
            needs_layout_passes=False, use_tc_tiling_on_sc=True),
        scratch_types=[
            pltpu.VMEM((b_per_w,), jnp.int32),
            pltpu.VMEM((b_per_w,), jnp.int32),
            pltpu.VMEM((n_chunks, IDX_CHUNK), jnp.int32),
            pltpu.VMEM((n_chunks, IDX_CHUNK), jnp.int32),
            pltpu.VMEM((n_chunks, IDX_CHUNK), jnp.int32),
            pltpu.VMEM((n_chunks, IDX_CHUNK), jnp.int32),
            pltpu.VMEM((b_per_w,), jnp.int32),
            pltpu.VMEM((b_per_w,), jnp.int32),
            pltpu.VMEM((2, IDX_CHUNK, wdim), jnp.float32),
            pltpu.VMEM((2, IDX_CHUNK, wdim), jnp.float32),
            pltpu.VMEM((b_per_w,), jnp.float32),
            pltpu.VMEM((b_per_w,), jnp.float32),
            pltpu.VMEM((b_per_w,), jnp.float32),
            pltpu.VMEM((b_per_w,), jnp.float32),
            pltpu.SemaphoreType.DMA,
            pltpu.SemaphoreType.DMA,
        ],
    )
    def mf(dense_hbm, uid_hbm, iid_hbm, pr_hbm, qr_hbm, ub_hbm, ib_hbm,
           out_hbm, uraw_v, iraw_v, upair_v, ipair_v, uw_v, iw_v, uoff_v,
           ioff_v, prows_v, qrows_v, dense_v, ub_v, ib_v, out_v, sem, bsem):
        wid = lax.axis_index("s") * NC + lax.axis_index("c")
        base = wid * b_per_w

        pltpu.sync_copy(uid_hbm.at[pl.ds(base, b_per_w)], uraw_v)
        pltpu.sync_copy(iid_hbm.at[pl.ds(base, b_per_w)], iraw_v)
        pltpu.sync_copy(dense_hbm.at[pl.ds(base, b_per_w)], dense_v)

        iota = lax.iota(jnp.int32, L)
        for j in range(b_per_w // L):
            sl = pl.ds(j * L, L)
            row, col = divmod(j * L, IDX_CHUNK)
            csl = pl.ds(col, L)
            u = uraw_v[sl]
            w = jnp.where(u == 0, n_users - 1, u - 1)
            j = w >> 12
            upair_v[row, csl] = ((j >> 1) << 12) + (w & 4095)
            uw_v[row, csl] = w
            uoff_v[sl] = (j & 1) << 6
            t = iraw_v[sl]
            w = jnp.where(t == 0, n_items - 1, t - 1)
            j = w >> 12
            ipair_v[row, csl] = ((j >> 1) << 12) + (w & 4095)
            iw_v[row, csl] = w
            ioff_v[sl] = (j & 1) << 6

        bias_copies = []
        for ck in range(n_chunks):
            sl = pl.ds(ck * IDX_CHUNK, IDX_CHUNK)
            bias_copies.append(
                pltpu.async_copy(ub_hbm.at[uw_v.at[ck]], ub_v.at[sl], bsem))
            bias_copies.append(
                pltpu.async_copy(ib_hbm.at[iw_v.at[ck]], ib_v.at[sl], bsem))

        def fire(ck, buf):
            cp = pltpu.async_copy(
                pr_hbm.at[upair_v.at[ck]], prows_v.at[buf], sem)
            cq = pltpu.async_copy(
                qr_hbm.at[ipair_v.at[ck]], qrows_v.at[buf], sem)
            return cp, cq

        inflight = fire(0, 0)
        for cp in bias_copies:
            cp.wait()

        for ck in range(n_chunks):
            buf = ck % 2
            cur = inflight
            if ck + 1 < n_chunks:
                inflight = fire(ck + 1, 1 - buf)
            cur[0].wait()
            cur[1].wait()

            def group(lg, carry):
                rid = iota + (ck * IDX_CHUNK + lg * L)
                lrid = iota + lg * L
                acc = (plsc.load_gather(dense_v, [rid])
                       + plsc.load_gather(ub_v, [rid])
                       + plsc.load_gather(ib_v, [rid]))
                uo = plsc.load_gather(uoff_v, [rid])
                io = plsc.load_gather(ioff_v, [rid])
                for d in range(dim):
                    pv = plsc.load_gather(prows_v, [jnp.full((L,), buf, jnp.int32),
                                                    lrid, uo + d])
                    qv = plsc.load_gather(qrows_v, [jnp.full((L,), buf, jnp.int32),
                                                    lrid, io + d])
                    acc = acc + pv * qv
                plsc.store_scatter(out_v, [rid], acc)
                return carry

            lax.fori_loop(0, groups_per_chunk, group, 0)

        pltpu.sync_copy(out_v, out_hbm.at[pl.ds(base, b_per_w)])

    return mf


def _build_pairify(n_rows, dim):
    blk = 4096
    grid = (n_rows + 2 * blk - 1) // (2 * blk)
    out_rows = grid * blk

    def body(xlo_ref, xhi_ref, o_ref):
        o_ref[:, 0:dim] = xlo_ref[...].T
        o_ref[:, dim:2 * dim] = xhi_ref[...].T

    return pl.pallas_call(
        body,
        grid=(grid,),
        in_specs=[pl.BlockSpec((dim, blk), lambda g: (0, 2 * g)),
                  pl.BlockSpec(
                      (dim, blk),
                      lambda g: (0, jnp.minimum(2 * g + 1, 2 * grid - 2)))],
        out_specs=pl.BlockSpec((blk, 2 * dim), lambda g: (g, 0)),
        out_shape=jax.ShapeDtypeStruct((out_rows, 2 * dim), jnp.float32),
    )


def kernel(dense_inputs, sparse_inputs, p, q, user_bias, item_bias):
    batch = sparse_inputs.shape[0]
    dim = p.shape[1]
    pairify = _build_pairify(p.shape[0], dim)
    pt = p.T
    qt = q.T
    pr = pairify(pt, pt)
    qr = pairify(qt, qt)
    mf = _build(batch, dim, p.shape[0], q.shape[0])
    out = mf(dense_inputs.reshape(-1),
             sparse_inputs[:, 0], sparse_inputs[:, 1],
             pr, qr,
             user_bias.reshape(-1), item_bias.reshape(-1))
    return out.reshape(batch, 1)


# speedup vs baseline: 2.0399x; 1.1088x over previous
import functools

import jax
import jax.numpy as jnp
from jax import lax
from jax.experimental import pallas as pl
from jax.experimental.pallas import tpu as pltpu
from jax.experimental.pallas import tpu_sc as plsc

NC = 2
NS = 16
NW = NC * NS
L = 16
IDX_CHUNK = 128


def _build(batch, dim, n_users, n_items):
    b_per_w = batch // NW
    n_chunks = b_per_w // IDX_CHUNK
    groups_per_chunk = IDX_CHUNK // L
    wdim = 2 * dim
    mesh = plsc.VectorSubcoreMesh(
        core_axis_name="c", subcore_axis_name="s", num_cores=NC, num_subcores=NS
    )

    @functools.partial(
        pl.kernel,
        mesh=mesh,
        out_type=jax.ShapeDtypeStruct((batch,), jnp.float32),
        compiler_params=pltpu.CompilerParams(
            needs_layout_passes=False, use_tc_tiling_on_sc=True),
        scratch_types=[
            pltpu.VMEM((b_per_w,), jnp.int32),
            pltpu.VMEM((b_per_w,), jnp.int32),
            pltpu.VMEM((n_chunks, IDX_CHUNK), jnp.int32),
            pltpu.VMEM((n_chunks, IDX_CHUNK), jnp.int32),
            pltpu.VMEM((n_chunks, IDX_CHUNK), jnp.int32),
            pltpu.VMEM((n_chunks, IDX_CHUNK), jnp.int32),
            pltpu.VMEM((b_per_w,), jnp.int32),
            pltpu.VMEM((b_per_w,), jnp.int32),
            pltpu.VMEM((2, IDX_CHUNK, wdim), jnp.float32),
            pltpu.VMEM((2, IDX_CHUNK, wdim), jnp.float32),
            pltpu.VMEM((b_per_w,), jnp.float32),
            pltpu.VMEM((b_per_w,), jnp.float32),
            pltpu.VMEM((b_per_w,), jnp.float32),
            pltpu.VMEM((b_per_w,), jnp.float32),
            pltpu.SemaphoreType.DMA,
            pltpu.SemaphoreType.DMA,
        ],
    )
    def mf(dense_hbm, uid_hbm, iid_hbm, pr_hbm, qr_hbm, ub_hbm, ib_hbm,
           out_hbm, uraw_v, iraw_v, upair_v, ipair_v, uw_v, iw_v, uoff_v,
           ioff_v, prows_v, qrows_v, dense_v, ub_v, ib_v, out_v, sem, bsem):
        wid = lax.axis_index("s") * NC + lax.axis_index("c")
        base = wid * b_per_w

        pltpu.sync_copy(uid_hbm.at[pl.ds(base, b_per_w)], uraw_v)
        pltpu.sync_copy(iid_hbm.at[pl.ds(base, b_per_w)], iraw_v)
        pltpu.sync_copy(dense_hbm.at[pl.ds(base, b_per_w)], dense_v)

        iota = lax.iota(jnp.int32, L)
        for j in range(b_per_w // L):
            sl = pl.ds(j * L, L)
            row, col = divmod(j * L, IDX_CHUNK)
            csl = pl.ds(col, L)
            u = uraw_v[sl]
            w = jnp.where(u == 0, n_users - 1, u - 1)
            j = w >> 13
            upair_v[row, csl] = ((j >> 1) << 13) + (w & 8191)
            uw_v[row, csl] = w
            uoff_v[sl] = (j & 1) << 6
            t = iraw_v[sl]
            w = jnp.where(t == 0, n_items - 1, t - 1)
            j = w >> 13
            ipair_v[row, csl] = ((j >> 1) << 13) + (w & 8191)
            iw_v[row, csl] = w
            ioff_v[sl] = (j & 1) << 6

        bias_copies = []
        for ck in range(n_chunks):
            sl = pl.ds(ck * IDX_CHUNK, IDX_CHUNK)
            bias_copies.append(
                pltpu.async_copy(ub_hbm.at[uw_v.at[ck]], ub_v.at[sl], bsem))
            bias_copies.append(
                pltpu.async_copy(ib_hbm.at[iw_v.at[ck]], ib_v.at[sl], bsem))

        def fire(ck, buf):
            cp = pltpu.async_copy(
                pr_hbm.at[upair_v.at[ck]], prows_v.at[buf], sem)
            cq = pltpu.async_copy(
                qr_hbm.at[ipair_v.at[ck]], qrows_v.at[buf], sem)
            return cp, cq

        inflight = fire(0, 0)
        for cp in bias_copies:
            cp.wait()

        for ck in range(n_chunks):
            buf = ck % 2
            cur = inflight
            if ck + 1 < n_chunks:
                inflight = fire(ck + 1, 1 - buf)
            cur[0].wait()
            cur[1].wait()

            def group(lg, carry):
                rid = iota + (ck * IDX_CHUNK + lg * L)
                lrid = iota + lg * L
                acc = (plsc.load_gather(dense_v, [rid])
                       + plsc.load_gather(ub_v, [rid])
                       + plsc.load_gather(ib_v, [rid]))
                uo = plsc.load_gather(uoff_v, [rid])
                io = plsc.load_gather(ioff_v, [rid])
                for d in range(dim):
                    pv = plsc.load_gather(prows_v, [jnp.full((L,), buf, jnp.int32),
                                                    lrid, uo + d])
                    qv = plsc.load_gather(qrows_v, [jnp.full((L,), buf, jnp.int32),
                                                    lrid, io + d])
                    acc = acc + pv * qv
                plsc.store_scatter(out_v, [rid], acc)
                return carry

            lax.fori_loop(0, groups_per_chunk, group, 0)

        pltpu.sync_copy(out_v, out_hbm.at[pl.ds(base, b_per_w)])

    return mf


def _build_pairify(n_rows, dim):
    blk = 8192
    grid = (n_rows + 2 * blk - 1) // (2 * blk)
    out_rows = grid * blk

    def body(xlo_ref, xhi_ref, o_ref):
        o_ref[:, 0:dim] = xlo_ref[...].T
        o_ref[:, dim:2 * dim] = xhi_ref[...].T

    return pl.pallas_call(
        body,
        grid=(grid,),
        in_specs=[pl.BlockSpec((dim, blk), lambda g: (0, 2 * g)),
                  pl.BlockSpec(
                      (dim, blk),
                      lambda g: (0, jnp.minimum(2 * g + 1, 2 * grid - 2)))],
        out_specs=pl.BlockSpec((blk, 2 * dim), lambda g: (g, 0)),
        out_shape=jax.ShapeDtypeStruct((out_rows, 2 * dim), jnp.float32),
    )


def kernel(dense_inputs, sparse_inputs, p, q, user_bias, item_bias):
    batch = sparse_inputs.shape[0]
    dim = p.shape[1]
    pairify = _build_pairify(p.shape[0], dim)
    pt = p.T
    qt = q.T
    pr = pairify(pt, pt)
    qr = pairify(qt, qt)
    mf = _build(batch, dim, p.shape[0], q.shape[0])
    out = mf(dense_inputs.reshape(-1),
             sparse_inputs[:, 0], sparse_inputs[:, 1],
             pr, qr,
             user_bias.reshape(-1), item_bias.reshape(-1))
    return out.reshape(batch, 1)


# speedup vs baseline: 2.1421x; 1.0501x over previous
import functools

import jax
import jax.numpy as jnp
from jax import lax
from jax.experimental import pallas as pl
from jax.experimental.pallas import tpu as pltpu
from jax.experimental.pallas import tpu_sc as plsc

NC = 2
NS = 16
NW = NC * NS
L = 16
IDX_CHUNK = 128


def _build(batch, dim, n_users, n_items):
    b_per_w = batch // NW
    n_chunks = b_per_w // IDX_CHUNK
    groups_per_chunk = IDX_CHUNK // L
    wdim = 2 * dim
    mesh = plsc.VectorSubcoreMesh(
        core_axis_name="c", subcore_axis_name="s", num_cores=NC, num_subcores=NS
    )

    @functools.partial(
        pl.kernel,
        mesh=mesh,
        out_type=jax.ShapeDtypeStruct((batch,), jnp.float32),
        compiler_params=pltpu.CompilerParams(
            needs_layout_passes=False, use_tc_tiling_on_sc=True),
        scratch_types=[
            pltpu.VMEM((b_per_w,), jnp.int32),
            pltpu.VMEM((b_per_w,), jnp.int32),
            pltpu.VMEM((n_chunks, IDX_CHUNK), jnp.int32),
            pltpu.VMEM((n_chunks, IDX_CHUNK), jnp.int32),
            pltpu.VMEM((n_chunks, IDX_CHUNK), jnp.int32),
            pltpu.VMEM((n_chunks, IDX_CHUNK), jnp.int32),
            pltpu.VMEM((b_per_w,), jnp.int32),
            pltpu.VMEM((b_per_w,), jnp.int32),
            pltpu.VMEM((2, IDX_CHUNK, wdim), jnp.float32),
            pltpu.VMEM((2, IDX_CHUNK, wdim), jnp.float32),
            pltpu.VMEM((b_per_w,), jnp.float32),
            pltpu.VMEM((b_per_w,), jnp.float32),
            pltpu.VMEM((b_per_w,), jnp.float32),
            pltpu.VMEM((b_per_w,), jnp.float32),
            pltpu.SemaphoreType.DMA,
            pltpu.SemaphoreType.DMA,
        ],
    )
    def mf(dense_hbm, uid_hbm, iid_hbm, pr_hbm, qr_hbm, ub_hbm, ib_hbm,
           out_hbm, uraw_v, iraw_v, upair_v, ipair_v, uw_v, iw_v, uoff_v,
           ioff_v, prows_v, qrows_v, dense_v, ub_v, ib_v, out_v, sem, bsem):
        wid = lax.axis_index("s") * NC + lax.axis_index("c")
        base = wid * b_per_w

        pltpu.sync_copy(uid_hbm.at[pl.ds(base, b_per_w)], uraw_v)
        pltpu.sync_copy(iid_hbm.at[pl.ds(base, b_per_w)], iraw_v)
        pltpu.sync_copy(dense_hbm.at[pl.ds(base, b_per_w)], dense_v)

        iota = lax.iota(jnp.int32, L)
        for j in range(b_per_w // L):
            sl = pl.ds(j * L, L)
            row, col = divmod(j * L, IDX_CHUNK)
            csl = pl.ds(col, L)
            u = uraw_v[sl]
            w = jnp.where(u == 0, n_users - 1, u - 1)
            j = w >> 14
            upair_v[row, csl] = ((j >> 1) << 14) + (w & 16383)
            uw_v[row, csl] = w
            uoff_v[sl] = (j & 1) << 6
            t = iraw_v[sl]
            w = jnp.where(t == 0, n_items - 1, t - 1)
            j = w >> 14
            ipair_v[row, csl] = ((j >> 1) << 14) + (w & 16383)
            iw_v[row, csl] = w
            ioff_v[sl] = (j & 1) << 6

        bias_copies = []
        for ck in range(n_chunks):
            sl = pl.ds(ck * IDX_CHUNK, IDX_CHUNK)
            bias_copies.append(
                pltpu.async_copy(ub_hbm.at[uw_v.at[ck]], ub_v.at[sl], bsem))
            bias_copies.append(
                pltpu.async_copy(ib_hbm.at[iw_v.at[ck]], ib_v.at[sl], bsem))

        def fire(ck, buf):
            cp = pltpu.async_copy(
                pr_hbm.at[upair_v.at[ck]], prows_v.at[buf], sem)
            cq = pltpu.async_copy(
                qr_hbm.at[ipair_v.at[ck]], qrows_v.at[buf], sem)
            return cp, cq

        inflight = fire(0, 0)
        for cp in bias_copies:
            cp.wait()

        for ck in range(n_chunks):
            buf = ck % 2
            cur = inflight
            if ck + 1 < n_chunks:
                inflight = fire(ck + 1, 1 - buf)
            cur[0].wait()
            cur[1].wait()

            def group(lg, carry):
                rid = iota + (ck * IDX_CHUNK + lg * L)
                lrid = iota + lg * L
                acc = (plsc.load_gather(dense_v, [rid])
                       + plsc.load_gather(ub_v, [rid])
                       + plsc.load_gather(ib_v, [rid]))
                uo = plsc.load_gather(uoff_v, [rid])
                io = plsc.load_gather(ioff_v, [rid])
                for d in range(dim):
                    pv = plsc.load_gather(prows_v, [jnp.full((L,), buf, jnp.int32),
                                                    lrid, uo + d])
                    qv = plsc.load_gather(qrows_v, [jnp.full((L,), buf, jnp.int32),
                                                    lrid, io + d])
                    acc = acc + pv * qv
                plsc.store_scatter(out_v, [rid], acc)
                return carry

            lax.fori_loop(0, groups_per_chunk, group, 0)

        pltpu.sync_copy(out_v, out_hbm.at[pl.ds(base, b_per_w)])

    return mf


def _build_pairify(n_rows, dim):
    blk = 16384
    grid = (n_rows + 2 * blk - 1) // (2 * blk)
    out_rows = grid * blk

    def body(xlo_ref, xhi_ref, o_ref):
        o_ref[:, 0:dim] = xlo_ref[...].T
        o_ref[:, dim:2 * dim] = xhi_ref[...].T

    return pl.pallas_call(
        body,
        grid=(grid,),
        in_specs=[pl.BlockSpec((dim, blk), lambda g: (0, 2 * g)),
                  pl.BlockSpec(
                      (dim, blk),
                      lambda g: (0, jnp.minimum(2 * g + 1, (n_rows - 1) // blk)))],
        out_specs=pl.BlockSpec((blk, 2 * dim), lambda g: (g, 0)),
        out_shape=jax.ShapeDtypeStruct((out_rows, 2 * dim), jnp.float32),
    )


def kernel(dense_inputs, sparse_inputs, p, q, user_bias, item_bias):
    batch = sparse_inputs.shape[0]
    dim = p.shape[1]
    pairify = _build_pairify(p.shape[0], dim)
    pt = p.T
    qt = q.T
    pr = pairify(pt, pt)
    qr = pairify(qt, qt)
    mf = _build(batch, dim, p.shape[0], q.shape[0])
    out = mf(dense_inputs.reshape(-1),
             sparse_inputs[:, 0], sparse_inputs[:, 1],
             pr, qr,
             user_bias.reshape(-1), item_bias.reshape(-1))
    return out.reshape(batch, 1)
